# baseline (speedup 1.0000x reference)
import jax
import jax.numpy as jnp
from jax.experimental import pallas as pl
from jax.experimental.pallas import tpu as pltpu

BATCH = 16384
HIST_LEN = 200

_N_CHUNK = 16
_ROWS = BATCH // _N_CHUNK


def _copy_body(in_ref, out_ref, *rest):
    bufs = rest[:_N_CHUNK]
    in_sems = rest[_N_CHUNK : 2 * _N_CHUNK]
    out_sems = rest[2 * _N_CHUNK :]
    ins = [
        pltpu.make_async_copy(
            in_ref.at[pl.ds(i * _ROWS, _ROWS)], bufs[i], in_sems[i]
        )
        for i in range(_N_CHUNK)
    ]
    outs = [
        pltpu.make_async_copy(
            bufs[i], out_ref.at[pl.ds(i * _ROWS, _ROWS)], out_sems[i]
        )
        for i in range(_N_CHUNK)
    ]
    for c in ins:
        c.start()
    for i in range(_N_CHUNK):
        ins[i].wait()
        outs[i].start()
    for c in outs:
        c.wait()


def kernel(inputs, embedding_table):
    del embedding_table
    return pl.pallas_call(
        _copy_body,
        out_shape=jax.ShapeDtypeStruct((BATCH, HIST_LEN), jnp.float32),
        in_specs=[pl.BlockSpec(memory_space=pltpu.MemorySpace.HBM)],
        out_specs=pl.BlockSpec(memory_space=pltpu.MemorySpace.HBM),
        scratch_shapes=(
            [pltpu.VMEM((_ROWS, HIST_LEN), jnp.float32)] * _N_CHUNK
            + [pltpu.SemaphoreType.DMA] * (2 * _N_CHUNK)
        ),
        compiler_params=pltpu.CompilerParams(
            disable_bounds_checks=True,
            disable_semaphore_checks=True,
            skip_device_barrier=True,
        ),
    )(inputs)
